# baseline (speedup 1.0000x reference)
import functools

import jax
import jax.numpy as jnp
from jax import lax
from jax.experimental import pallas as pl
from jax.experimental.pallas import tpu as pltpu
from jax.experimental.pallas import tpu_sc as plsc

B = 16384
D = 64
NC = 2
NS = 16
NW = NC * NS
BPW = B // NW
L = 16
DP = D + 1
EPS = 1e-5

_mesh = plsc.VectorSubcoreMesh(core_axis_name="c", subcore_axis_name="s")


@functools.partial(
    pl.kernel,
    mesh=_mesh,
    out_type=jax.ShapeDtypeStruct((D, B), jnp.float32),
    scratch_types=[
        pltpu.VMEM((BPW,), jnp.int32),
        pltpu.VMEM((BPW, DP), jnp.float32),
        pltpu.VMEM((D, BPW), jnp.float32),
        pltpu.VMEM((D,), jnp.float32),
        pltpu.VMEM((D,), jnp.float32),
        pltpu.SemaphoreType.DMA,
    ],
    compiler_params=pltpu.CompilerParams(needs_layout_passes=False),
)
def _embed_ln(ids_hbm, w_hbm, g_hbm, b_hbm, out_hbm, idx_v, rows_p, cols_v, g_v, b_v, sem):
    wid = lax.axis_index("s") * NC + lax.axis_index("c")
    base = wid * BPW

    pltpu.sync_copy(ids_hbm.at[pl.ds(base, BPW)], idx_v)
    pltpu.sync_copy(g_hbm, g_v)
    pltpu.sync_copy(b_hbm, b_v)

    def issue(g, carry):
        vec = idx_v[pl.ds(g * L, L)]
        for k in range(L):
            pltpu.async_copy(w_hbm.at[vec[k]], rows_p.at[g * L + k, pl.ds(0, D)], sem)
        return carry

    lax.fori_loop(0, BPW // L, issue, 0)
    pltpu.make_async_copy(out_hbm.at[:, pl.ds(0, BPW)], cols_v, sem).wait()

    gsc = [g_v[pl.ds((c // L) * L, L)][c % L] for c in range(D)]
    bsc = [b_v[pl.ds((c // L) * L, L)][c % L] for c in range(D)]

    iota = lax.iota(jnp.int32, L)

    def tree_sum(vs):
        while len(vs) > 1:
            vs = [a + b for a, b in zip(vs[0::2], vs[1::2])]
        return vs[0]

    @plsc.parallel_loop(0, BPW // L, step=1, unroll=2)
    def body(g):
        ridx = g * L + iota
        cols = [plsc.load_gather(rows_p, [ridx, jnp.full((L,), c, jnp.int32)])
                for c in range(D)]
        mean = tree_sum(list(cols)) * (1.0 / D)
        ds_ = [v - mean for v in cols]
        var = tree_sum([v * v for v in ds_]) * (1.0 / D)
        x = var + EPS
        xi = lax.bitcast_convert_type(x, jnp.int32)
        magic = jnp.full((L,), 0x5F3759DF, dtype=jnp.int32)
        y = lax.bitcast_convert_type(magic - (xi >> 1), jnp.float32)
        hx = x * -0.5
        for _ in range(3):
            y = y * (y * y * hx + 1.5)
        for c in range(D):
            cols_v[c, pl.ds(g * L, L)] = ds_[c] * (y * gsc[c]) + bsc[c]

    pltpu.sync_copy(cols_v, out_hbm.at[:, pl.ds(base, BPW)])


def kernel(input_ids, weight, gamma, beta):
    ids = input_ids.reshape(-1).astype(jnp.int32)
    out_t = _embed_ln(ids, weight, gamma, beta)
    return out_t.T.reshape(B, 1, D)
